# Initial kernel scaffold; baseline (speedup 1.0000x reference)
import jax
import jax.numpy as jnp
from jax.experimental import pallas as pl


def kernel(node_indices, positions, adjacency, mask, species_energy, pair_weight):
    raise NotImplementedError("write your pallas kernel here")


# speedup vs baseline: 2.1980x; 2.1980x over previous
import jax
import jax.numpy as jnp
from jax.experimental import pallas as pl

_TI = 256
_SP = 128


def _body(row_ref, col_ref, adj_ref, se_ref, pw_ref, out_ref):
    i = pl.program_id(1)

    xi = row_ref[0, :, 0:1]
    yi = row_ref[0, :, 1:2]
    zi = row_ref[0, :, 2:3]
    mi = row_ref[0, :, 3:4]
    ids = row_ref[0, :, 4:5]

    xj = col_ref[0, 0:1, :]
    yj = col_ref[0, 1:2, :]
    zj = col_ref[0, 2:3, :]
    mj = col_ref[0, 3:4, :]

    adj = adj_ref[0].astype(jnp.float32)

    dx = xi - xj
    dy = yi - yj
    dz = zi - zj
    d2 = dx * dx + dy * dy + dz * dz
    dist = jnp.sqrt(d2)
    pair = jnp.sum((adj * mj) * dist * mi)

    sp = jax.lax.broadcasted_iota(jnp.int32, (1, _SP), 1).astype(jnp.float32)
    oh = (ids == sp).astype(jnp.float32)
    atom = jnp.sum(oh * se_ref[0] * mi)

    e = atom + pw_ref[0, 0, 0] * pair

    @pl.when(i == 0)
    def _():
        out_ref[...] = jnp.zeros_like(out_ref)

    out_ref[...] += e


def kernel(node_indices, positions, adjacency, mask, species_energy, pair_weight):
    B, N = node_indices.shape
    S = species_energy.shape[0]
    ni = N // _TI

    maskf = mask.astype(jnp.float32)
    idsf = node_indices.astype(jnp.float32)

    rowpack = jnp.concatenate(
        [positions, maskf[:, :, None], idsf[:, :, None]], axis=-1
    )
    rowpack = jnp.pad(rowpack, ((0, 0), (0, 0), (0, 128 - 5)))

    colpack = jnp.concatenate(
        [positions.transpose(0, 2, 1), maskf[:, None, :]], axis=1
    )

    se_row = jnp.zeros((1, 1, _SP), jnp.float32).at[0, 0, :S].set(species_energy)
    pw_arr = jnp.full((1, 1, 128), pair_weight, jnp.float32)

    out = pl.pallas_call(
        _body,
        grid=(B, ni),
        in_specs=[
            pl.BlockSpec((1, _TI, 128), lambda b, i: (b, i, 0)),
            pl.BlockSpec((1, 4, N), lambda b, i: (b, 0, 0)),
            pl.BlockSpec((1, _TI, N), lambda b, i: (b, i, 0)),
            pl.BlockSpec((1, 1, _SP), lambda b, i: (0, 0, 0)),
            pl.BlockSpec((1, 1, 128), lambda b, i: (0, 0, 0)),
        ],
        out_specs=pl.BlockSpec((1, 1, 128), lambda b, i: (b, 0, 0)),
        out_shape=jax.ShapeDtypeStruct((B, 1, 128), jnp.float32),
    )(rowpack, colpack, adjacency, se_row, pw_arr)

    return out[:, 0, 0]


# speedup vs baseline: 2.8223x; 1.2840x over previous
import jax
import jax.numpy as jnp
from jax.experimental import pallas as pl

_TI = 256
_SP = 128


def _body(row_ref, col_ref, adj_ref, se_ref, out_ref):
    i = pl.program_id(1)

    ai = row_ref[0, :, 0:8]
    bj = col_ref[0, 0:8, :]
    d2 = jax.lax.dot_general(
        ai[:, 0:5], bj[0:5, :], (((1,), (0,)), ((), ())),
        preferred_element_type=jnp.float32,
    )
    d2c = jnp.maximum(d2, 1e-12)
    dist = d2c * jax.lax.rsqrt(d2c)
    w = adj_ref[0].astype(jnp.float32) * dist

    mi = row_ref[0, :, 5:6]
    t1 = jax.lax.dot_general(
        mi, w, (((0,), (0,)), ((), ())), preferred_element_type=jnp.float32
    )
    pair = jnp.sum(t1 * col_ref[0, 5:6, :])

    ids = row_ref[0, :, 6:7]
    sp = jax.lax.broadcasted_iota(jnp.int32, (1, _SP), 1).astype(jnp.float32)
    oh = (ids == sp).astype(jnp.float32)
    atom = jnp.sum(oh * se_ref[0])

    e = atom + pair

    @pl.when(i == 0)
    def _():
        out_ref[...] = jnp.zeros_like(out_ref)

    out_ref[...] += e


def kernel(node_indices, positions, adjacency, mask, species_energy, pair_weight):
    B, N = node_indices.shape
    S = species_energy.shape[0]
    ni = N // _TI

    maskf = mask.astype(jnp.float32)
    idsf = jnp.where(mask, node_indices, -1).astype(jnp.float32)
    r2 = jnp.sum(positions * positions, axis=-1, keepdims=True)
    onesc = jnp.ones_like(r2)

    rowpack = jnp.concatenate(
        [positions, r2, onesc, maskf[:, :, None], idsf[:, :, None]], axis=-1
    )
    rowpack = jnp.pad(rowpack, ((0, 0), (0, 0), (0, 128 - 7)))

    pw = pair_weight.astype(jnp.float32)
    colpack = jnp.concatenate(
        [
            -2.0 * positions.transpose(0, 2, 1),
            onesc.transpose(0, 2, 1),
            r2.transpose(0, 2, 1),
            pw * maskf[:, None, :],
            jnp.zeros((B, 2, N), jnp.float32),
        ],
        axis=1,
    )

    se_row = jnp.zeros((1, 1, _SP), jnp.float32).at[0, 0, :S].set(species_energy)

    out = pl.pallas_call(
        _body,
        grid=(B, ni),
        in_specs=[
            pl.BlockSpec((1, _TI, 128), lambda b, i: (b, i, 0)),
            pl.BlockSpec((1, 8, N), lambda b, i: (b, 0, 0)),
            pl.BlockSpec((1, _TI, N), lambda b, i: (b, i, 0)),
            pl.BlockSpec((1, 1, _SP), lambda b, i: (0, 0, 0)),
        ],
        out_specs=pl.BlockSpec((1, 1, 128), lambda b, i: (b, 0, 0)),
        out_shape=jax.ShapeDtypeStruct((B, 1, 128), jnp.float32),
    )(rowpack, colpack, adjacency, se_row)

    return out[:, 0, 0]


# speedup vs baseline: 3.7992x; 1.3462x over previous
import jax
import jax.numpy as jnp
from jax.experimental import pallas as pl

_C = 512
_SP = 128


def _body(row_ref, col_ref, adj_ref, se_ref, out_ref):
    N = adj_ref.shape[2]
    bj = col_ref[0, 0:5, :]
    colm = col_ref[0, 5:6, :]

    def chunk(c, t1):
        sl = pl.ds(c * _C, _C)
        ai = row_ref[0, sl, 0:5]
        d2 = jax.lax.dot_general(
            ai, bj, (((1,), (0,)), ((), ())),
            preferred_element_type=jnp.float32,
        )
        d2c = jnp.maximum(d2, 1e-12)
        dist = d2c * jax.lax.rsqrt(d2c)
        w = adj_ref[0, sl, :].astype(jnp.float32) * dist
        mi = row_ref[0, sl, 5:6]
        return t1 + jax.lax.dot_general(
            mi, w, (((0,), (0,)), ((), ())), preferred_element_type=jnp.float32
        )

    t1 = jax.lax.fori_loop(0, N // _C, chunk, jnp.zeros((1, N), jnp.float32))
    pair = jnp.sum(t1 * colm)

    ids = row_ref[0, :, 6:7]
    sp = jax.lax.broadcasted_iota(jnp.int32, (1, _SP), 1).astype(jnp.float32)
    oh = (ids == sp).astype(jnp.float32)
    atom = jnp.sum(oh * se_ref[0])

    out_ref[...] = jnp.full_like(out_ref, atom + pair)


def kernel(node_indices, positions, adjacency, mask, species_energy, pair_weight):
    B, N = node_indices.shape
    S = species_energy.shape[0]

    maskf = mask.astype(jnp.float32)
    idsf = jnp.where(mask, node_indices, -1).astype(jnp.float32)
    r2 = jnp.sum(positions * positions, axis=-1, keepdims=True)
    onesc = jnp.ones_like(r2)

    rowpack = jnp.concatenate(
        [positions, r2, onesc, maskf[:, :, None], idsf[:, :, None]], axis=-1
    )
    rowpack = jnp.pad(rowpack, ((0, 0), (0, 0), (0, 128 - 7)))

    pw = pair_weight.astype(jnp.float32)
    colpack = jnp.concatenate(
        [
            -2.0 * positions.transpose(0, 2, 1),
            onesc.transpose(0, 2, 1),
            r2.transpose(0, 2, 1),
            pw * maskf[:, None, :],
            jnp.zeros((B, 2, N), jnp.float32),
        ],
        axis=1,
    )

    se_row = jnp.zeros((1, 1, _SP), jnp.float32).at[0, 0, :S].set(species_energy)

    out = pl.pallas_call(
        _body,
        grid=(B,),
        in_specs=[
            pl.BlockSpec((1, N, 128), lambda b: (b, 0, 0)),
            pl.BlockSpec((1, 8, N), lambda b: (b, 0, 0)),
            pl.BlockSpec((1, N, N), lambda b: (b, 0, 0)),
            pl.BlockSpec((1, 1, _SP), lambda b: (0, 0, 0)),
        ],
        out_specs=pl.BlockSpec((1, 1, 128), lambda b: (b, 0, 0)),
        out_shape=jax.ShapeDtypeStruct((B, 1, 128), jnp.float32),
    )(rowpack, colpack, adjacency, se_row)

    return out[:, 0, 0]
